# baseline (speedup 1.0000x reference)
import functools

import jax
import jax.numpy as jnp
from jax import lax
from jax.experimental import pallas as pl
from jax.experimental.pallas import tpu as pltpu
from jax.experimental.pallas import tpu_sc as plsc

NOP = 100000
HIDDEN_DIM = 64
BATCH = 16384

_info = plsc.get_sparse_core_info()
_NC, _NS = _info.num_cores, _info.num_subcores
_NW = _NC * _NS
_ROWS_PER_W = HIDDEN_DIM // _NW
_SPLIT = 50048
_ALIGNED_END = 99968
_AUX0 = NOP - 128
_LN1 = _ALIGNED_END - _SPLIT


def _make_gather():
    mesh = plsc.VectorSubcoreMesh(core_axis_name="c", subcore_axis_name="s")

    @functools.partial(
        pl.kernel,
        mesh=mesh,
        out_type=jax.ShapeDtypeStruct((HIDDEN_DIM, BATCH), jnp.float32),
        scratch_types=[
            pltpu.VMEM((BATCH,), jnp.int32),
            pltpu.VMEM((1, _SPLIT), jnp.float32),
            pltpu.VMEM((1, BATCH), jnp.float32),
            pltpu.SemaphoreType.DMA,
        ],
        compiler_params=pltpu.CompilerParams(needs_layout_passes=False),
    )
    def gather_kernel(
        idx_hbm, tT_hbm, tailT_hbm, outT_hbm, idx_v, slab_v, orow_v, sem
    ):
        wid = lax.axis_index("s") * _NC + lax.axis_index("c")
        pltpu.sync_copy(idx_hbm, idx_v)
        lane = lax.iota(jnp.int32, 16)
        zero16 = jnp.zeros((16,), jnp.int32)

        for hp in range(_ROWS_PER_W):
            h = _ROWS_PER_W * wid + hp

            pltpu.sync_copy(
                tT_hbm.at[pl.ds(h, 1), pl.ds(0, _SPLIT)],
                slab_v,
            )

            def step0(k, _):
                iv = idx_v[pl.ds(k * 16, 16)]
                m = iv < _SPLIT
                g = plsc.load_gather(slab_v, [zero16, iv], mask=m)
                plsc.store_scatter(orow_v, [zero16, lane + k * 16], g, mask=m)
                return ()

            lax.fori_loop(0, BATCH // 16, step0, (), unroll=4)

            pltpu.sync_copy(
                tT_hbm.at[pl.ds(h, 1), pl.ds(_SPLIT, _LN1)],
                slab_v.at[:, pl.ds(0, _LN1)],
            )
            pltpu.sync_copy(
                tailT_hbm.at[pl.ds(h, 1), :],
                slab_v.at[:, pl.ds(_LN1, 128)],
            )

            def step1(k, _):
                iv = idx_v[pl.ds(k * 16, 16)]
                m = iv >= _SPLIT
                rel = jnp.where(
                    iv >= _AUX0, iv - (_AUX0 - _LN1), iv - _SPLIT
                )
                g = plsc.load_gather(slab_v, [zero16, rel], mask=m)
                plsc.store_scatter(orow_v, [zero16, lane + k * 16], g, mask=m)
                return ()

            lax.fori_loop(0, BATCH // 16, step1, (), unroll=4)

            pltpu.sync_copy(orow_v, outT_hbm.at[pl.ds(h, 1), :])

    return gather_kernel


_gather = _make_gather()


def kernel(problem_id, embedding_table):
    tail_t = embedding_table[_AUX0:].T
    out_t = _gather(problem_id, embedding_table.T, tail_t)
    return out_t.T


# speedup vs baseline: 1.9386x; 1.9386x over previous
import functools

import jax
import jax.numpy as jnp
from jax import lax
from jax.experimental import pallas as pl
from jax.experimental.pallas import tpu as pltpu
from jax.experimental.pallas import tpu_sc as plsc

NOP = 100000
HIDDEN_DIM = 64
BATCH = 16384

_info = plsc.get_sparse_core_info()
_NC, _NS = _info.num_cores, _info.num_subcores
_NW = _NC * _NS
_ROWS_PER_W = HIDDEN_DIM // _NW
_SPLIT = 50048
_ALIGNED_END = 99968
_AUX0 = NOP - 128
_LN1 = _ALIGNED_END - _SPLIT


def _make_gather():
    mesh = plsc.VectorSubcoreMesh(core_axis_name="c", subcore_axis_name="s")

    @functools.partial(
        pl.kernel,
        mesh=mesh,
        out_type=jax.ShapeDtypeStruct((HIDDEN_DIM, BATCH), jnp.float32),
        scratch_types=[
            pltpu.VMEM((BATCH,), jnp.int32),
            pltpu.VMEM((1, _SPLIT), jnp.float32),
            pltpu.VMEM((1, BATCH), jnp.float32),
            pltpu.SemaphoreType.DMA,
        ],
        compiler_params=pltpu.CompilerParams(needs_layout_passes=False),
    )
    def gather_kernel(
        idx_hbm, tT_hbm, tailT_hbm, outT_hbm, idx_v, slab_v, orow_v, sem
    ):
        wid = lax.axis_index("s") * _NC + lax.axis_index("c")
        pltpu.sync_copy(idx_hbm, idx_v)
        lane = lax.iota(jnp.int32, 16)
        zero16 = jnp.zeros((16,), jnp.int32)

        for hp in range(_ROWS_PER_W):
            h = _ROWS_PER_W * wid + hp

            pltpu.sync_copy(
                tT_hbm.at[pl.ds(h, 1), pl.ds(0, _SPLIT)],
                slab_v,
            )

            @plsc.parallel_loop(0, BATCH, step=16, unroll=8)
            def step0(k):
                iv = idx_v[pl.ds(k, 16)]
                m = iv < _SPLIT
                g = plsc.load_gather(slab_v, [zero16, iv], mask=m)
                plsc.store_scatter(orow_v, [zero16, lane + k], g, mask=m)

            pltpu.sync_copy(
                tT_hbm.at[pl.ds(h, 1), pl.ds(_SPLIT, _LN1)],
                slab_v.at[:, pl.ds(0, _LN1)],
            )
            pltpu.sync_copy(
                tailT_hbm.at[pl.ds(h, 1), :],
                slab_v.at[:, pl.ds(_LN1, 128)],
            )

            @plsc.parallel_loop(0, BATCH, step=16, unroll=8)
            def step1(k):
                iv = idx_v[pl.ds(k, 16)]
                m = iv >= _SPLIT
                rel = jnp.where(
                    iv >= _AUX0, iv - (_AUX0 - _LN1), iv - _SPLIT
                )
                g = plsc.load_gather(slab_v, [zero16, rel], mask=m)
                plsc.store_scatter(orow_v, [zero16, lane + k], g, mask=m)

            pltpu.sync_copy(orow_v, outT_hbm.at[pl.ds(h, 1), :])

    return gather_kernel


_gather = _make_gather()


def kernel(problem_id, embedding_table):
    tail_t = embedding_table[_AUX0:].T
    out_t = _gather(problem_id, embedding_table.T, tail_t)
    return out_t.T


# speedup vs baseline: 1.9424x; 1.0019x over previous
import functools

import jax
import jax.numpy as jnp
from jax import lax
from jax.experimental import pallas as pl
from jax.experimental.pallas import tpu as pltpu
from jax.experimental.pallas import tpu_sc as plsc

NOP = 100000
HIDDEN_DIM = 64
BATCH = 16384

_info = plsc.get_sparse_core_info()
_NC, _NS = _info.num_cores, _info.num_subcores
_NW = _NC * _NS
_ROWS_PER_W = HIDDEN_DIM // _NW
_SPLIT = 50048
_ALIGNED_END = 99968
_AUX0 = NOP - 128
_LN1 = _ALIGNED_END - _SPLIT


def _make_gather():
    mesh = plsc.VectorSubcoreMesh(core_axis_name="c", subcore_axis_name="s")

    @functools.partial(
        pl.kernel,
        mesh=mesh,
        out_type=jax.ShapeDtypeStruct((HIDDEN_DIM, BATCH), jnp.float32),
        scratch_types=[
            pltpu.VMEM((BATCH,), jnp.int32),
            pltpu.VMEM((1, _SPLIT), jnp.float32),
            pltpu.VMEM((1, BATCH), jnp.float32),
            pltpu.SemaphoreType.DMA,
        ],
        compiler_params=pltpu.CompilerParams(
            needs_layout_passes=False, skip_device_barrier=True
        ),
    )
    def gather_kernel(
        idx_hbm, tT_hbm, tailT_hbm, outT_hbm, idx_v, slab_v, orow_v, sem
    ):
        wid = lax.axis_index("s") * _NC + lax.axis_index("c")
        pltpu.sync_copy(idx_hbm, idx_v)
        lane = lax.iota(jnp.int32, 16)
        zero16 = jnp.zeros((16,), jnp.int32)

        for hp in range(_ROWS_PER_W):
            h = _ROWS_PER_W * wid + hp

            pltpu.sync_copy(
                tT_hbm.at[pl.ds(h, 1), pl.ds(0, _SPLIT)],
                slab_v,
            )

            @plsc.parallel_loop(0, BATCH, step=16, unroll=8)
            def step0(k):
                iv = idx_v[pl.ds(k, 16)]
                m = iv < _SPLIT
                g = plsc.load_gather(slab_v, [zero16, iv], mask=m)
                plsc.store_scatter(orow_v, [zero16, lane + k], g, mask=m)

            pltpu.sync_copy(
                tT_hbm.at[pl.ds(h, 1), pl.ds(_SPLIT, _LN1)],
                slab_v.at[:, pl.ds(0, _LN1)],
            )
            pltpu.sync_copy(
                tailT_hbm.at[pl.ds(h, 1), :],
                slab_v.at[:, pl.ds(_LN1, 128)],
            )

            @plsc.parallel_loop(0, BATCH, step=16, unroll=8)
            def step1(k):
                iv = idx_v[pl.ds(k, 16)]
                m = iv >= _SPLIT
                rel = jnp.where(
                    iv >= _AUX0, iv - (_AUX0 - _LN1), iv - _SPLIT
                )
                g = plsc.load_gather(slab_v, [zero16, rel], mask=m)
                plsc.store_scatter(orow_v, [zero16, lane + k], g, mask=m)

            pltpu.sync_copy(orow_v, outT_hbm.at[pl.ds(h, 1), :])

    return gather_kernel


_gather = _make_gather()


def kernel(problem_id, embedding_table):
    tail_t = embedding_table[_AUX0:].T
    out_t = _gather(problem_id, embedding_table.T, tail_t)
    return out_t.T
